# baseline (speedup 1.0000x reference)
import functools

import jax
import jax.numpy as jnp
from jax import lax
from jax.experimental import pallas as pl
from jax.experimental.pallas import tpu as pltpu
from jax.experimental.pallas import tpu_sc as plsc


def _embed_lookup(ids, tok_embed, pos_embed):
    batch, seq_len = ids.shape
    B = batch * seq_len
    _, d = tok_embed.shape
    info = plsc.get_sparse_core_info()
    num_workers = info.num_cores * info.num_subcores
    s_per_w = seq_len // num_workers
    ch = s_per_w // 2
    nch = batch * 2
    depth = 3
    mesh = plsc.VectorSubcoreMesh(core_axis_name="c", subcore_axis_name="s")

    @functools.partial(
        pl.kernel,
        mesh=mesh,
        out_type=jax.ShapeDtypeStruct((B, d), jnp.float32),
        scratch_types=[
            pltpu.VMEM((batch, s_per_w), jnp.int32),
            pltpu.VMEM((nch, ch, d), jnp.float32),
            pltpu.VMEM((s_per_w, d), jnp.float32),
            pltpu.SemaphoreType.DMA((nch,)),
        ],
    )
    def _emb(ids_hbm, tok_hbm, pos_hbm, out_hbm, idx_v, tok_v, pos_v, sem):
        wid = lax.axis_index("s") * info.num_cores + lax.axis_index("c")
        sbase = pl.multiple_of(wid * s_per_w, s_per_w)

        idx_copies = [
            pltpu.async_copy(ids_hbm.at[b, pl.ds(sbase, s_per_w)],
                             idx_v.at[b], sem.at[b])
            for b in range(batch)
        ]
        pos_copy = pltpu.async_copy(pos_hbm.at[pl.ds(sbase, s_per_w)],
                                    pos_v, sem.at[batch])
        for c in idx_copies:
            c.wait()
        pos_copy.wait()

        def prefill(c):
            h = c - 2 * (c // 2)

            @plsc.parallel_loop(0, ch, unroll=2)
            def _row(i):
                for j in range(d // 16):
                    sl = pl.ds(j * 16, 16)
                    tok_v[c, i, sl] = pos_v[h * ch + i, sl]

        def gather(c):
            b = c // 2
            h = c - 2 * b
            return pltpu.make_async_copy(
                tok_hbm.at[idx_v.at[b, pl.ds(h * ch, ch)]],
                tok_v.at[c], sem.at[c])

        def store(c):
            b = c // 2
            h = c - 2 * b
            return pltpu.make_async_copy(
                tok_v.at[c],
                out_hbm.at[pl.ds(b * seq_len + sbase + h * ch, ch)],
                sem.at[c])

        def fire(c, x):
            prefill(c)
            gather(c).start(add=True)
            return x

        lax.fori_loop(0, depth, fire, 0)

        def consume(c, _):
            gather(c).wait()
            store(c).start()

            @pl.when(c + depth < nch)
            def _fire_next():
                fire(c + depth, 0)

            return _

        lax.fori_loop(0, nch, consume, 0)

        lax.fori_loop(0, nch, lambda c, x: (store(c).wait(), x)[1], 0)

    return _emb(ids, tok_embed, pos_embed)


def kernel(ids, tok_embed, pos_embed):
    batch, seq_len = ids.shape
    _, d = tok_embed.shape
    out = _embed_lookup(ids.astype(jnp.int32), tok_embed, pos_embed)
    return out.reshape(batch, seq_len, d)


# speedup vs baseline: 1.0299x; 1.0299x over previous
import functools

import jax
import jax.numpy as jnp
from jax import lax
from jax.experimental import pallas as pl
from jax.experimental.pallas import tpu as pltpu
from jax.experimental.pallas import tpu_sc as plsc


def _embed_lookup(ids, tok_embed, pos_embed):
    batch, seq_len = ids.shape
    B = batch * seq_len
    _, d = tok_embed.shape
    info = plsc.get_sparse_core_info()
    num_workers = info.num_cores * info.num_subcores
    s_per_w = seq_len // num_workers
    ch = s_per_w // 2
    nch = batch * 2
    depth = 4
    mesh = plsc.VectorSubcoreMesh(core_axis_name="c", subcore_axis_name="s")

    @functools.partial(
        pl.kernel,
        mesh=mesh,
        out_type=jax.ShapeDtypeStruct((B, d), jnp.float32),
        scratch_types=[
            pltpu.VMEM((batch, s_per_w), jnp.int32),
            pltpu.VMEM((nch, ch, d), jnp.float32),
            pltpu.VMEM((s_per_w, d), jnp.float32),
            pltpu.SemaphoreType.DMA((nch,)),
        ],
    )
    def _emb(ids_hbm, tok_hbm, pos_hbm, out_hbm, idx_v, tok_v, pos_v, sem):
        wid = lax.axis_index("s") * info.num_cores + lax.axis_index("c")
        sbase = pl.multiple_of(wid * s_per_w, s_per_w)

        idx_copies = [
            pltpu.async_copy(ids_hbm.at[b, pl.ds(sbase, s_per_w)],
                             idx_v.at[b], sem.at[b])
            for b in range(batch)
        ]
        pos_copy = pltpu.async_copy(pos_hbm.at[pl.ds(sbase, s_per_w)],
                                    pos_v, sem.at[batch])
        for c in idx_copies:
            c.wait()
        pos_copy.wait()

        def prefill(c):
            h = c - 2 * (c // 2)

            @plsc.parallel_loop(0, ch, unroll=2)
            def _row(i):
                for j in range(d // 16):
                    sl = pl.ds(j * 16, 16)
                    tok_v[c, i, sl] = pos_v[h * ch + i, sl]

        def gather(c):
            b = c // 2
            h = c - 2 * b
            return pltpu.make_async_copy(
                tok_hbm.at[idx_v.at[b, pl.ds(h * ch, ch)]],
                tok_v.at[c], sem.at[c])

        def store(c):
            b = c // 2
            h = c - 2 * b
            return pltpu.make_async_copy(
                tok_v.at[c],
                out_hbm.at[pl.ds(b * seq_len + sbase + h * ch, ch)],
                sem.at[c])

        def fire(c, x):
            prefill(c)
            gather(c).start(add=True)
            return x

        lax.fori_loop(0, depth, fire, 0)

        def consume(c, _):
            gather(c).wait()
            store(c).start()

            @pl.when(c + depth < nch)
            def _fire_next():
                fire(c + depth, 0)

            return _

        lax.fori_loop(0, nch, consume, 0)

        lax.fori_loop(0, nch, lambda c, x: (store(c).wait(), x)[1], 0)

    return _emb(ids, tok_embed, pos_embed)


def kernel(ids, tok_embed, pos_embed):
    batch, seq_len = ids.shape
    _, d = tok_embed.shape
    out = _embed_lookup(ids.astype(jnp.int32), tok_embed, pos_embed)
    return out.reshape(batch, seq_len, d)


# speedup vs baseline: 1.0374x; 1.0073x over previous
import functools

import jax
import jax.numpy as jnp
from jax import lax
from jax.experimental import pallas as pl
from jax.experimental.pallas import tpu as pltpu
from jax.experimental.pallas import tpu_sc as plsc


def _embed_lookup(ids, tok_embed, pos_embed):
    batch, seq_len = ids.shape
    B = batch * seq_len
    _, d = tok_embed.shape
    info = plsc.get_sparse_core_info()
    num_workers = info.num_cores * info.num_subcores
    s_per_w = seq_len // num_workers
    ch = s_per_w // 2
    nch = batch * 2
    depth = 5
    mesh = plsc.VectorSubcoreMesh(core_axis_name="c", subcore_axis_name="s")

    @functools.partial(
        pl.kernel,
        mesh=mesh,
        out_type=jax.ShapeDtypeStruct((B, d), jnp.float32),
        scratch_types=[
            pltpu.VMEM((batch, s_per_w), jnp.int32),
            pltpu.VMEM((nch, ch, d), jnp.float32),
            pltpu.VMEM((s_per_w, d), jnp.float32),
            pltpu.SemaphoreType.DMA((nch,)),
        ],
    )
    def _emb(ids_hbm, tok_hbm, pos_hbm, out_hbm, idx_v, tok_v, pos_v, sem):
        wid = lax.axis_index("s") * info.num_cores + lax.axis_index("c")
        sbase = pl.multiple_of(wid * s_per_w, s_per_w)

        idx_copies = [
            pltpu.async_copy(ids_hbm.at[b, pl.ds(sbase, s_per_w)],
                             idx_v.at[b], sem.at[b])
            for b in range(batch)
        ]
        pos_copy = pltpu.async_copy(pos_hbm.at[pl.ds(sbase, s_per_w)],
                                    pos_v, sem.at[batch])
        for c in idx_copies:
            c.wait()
        pos_copy.wait()

        def prefill(c):
            h = c - 2 * (c // 2)

            @plsc.parallel_loop(0, ch, unroll=2)
            def _row(i):
                for j in range(d // 16):
                    sl = pl.ds(j * 16, 16)
                    tok_v[c, i, sl] = pos_v[h * ch + i, sl]

        def gather(c):
            b = c // 2
            h = c - 2 * b
            return pltpu.make_async_copy(
                tok_hbm.at[idx_v.at[b, pl.ds(h * ch, ch)]],
                tok_v.at[c], sem.at[c])

        def store(c):
            b = c // 2
            h = c - 2 * b
            return pltpu.make_async_copy(
                tok_v.at[c],
                out_hbm.at[pl.ds(b * seq_len + sbase + h * ch, ch)],
                sem.at[c])

        def fire(c, x):
            prefill(c)
            gather(c).start(add=True)
            return x

        lax.fori_loop(0, depth, fire, 0)

        def consume(c, _):
            gather(c).wait()
            store(c).start()

            @pl.when(c + depth < nch)
            def _fire_next():
                fire(c + depth, 0)

            return _

        lax.fori_loop(0, nch, consume, 0)

        lax.fori_loop(0, nch, lambda c, x: (store(c).wait(), x)[1], 0)

    return _emb(ids, tok_embed, pos_embed)


def kernel(ids, tok_embed, pos_embed):
    batch, seq_len = ids.shape
    _, d = tok_embed.shape
    out = _embed_lookup(ids.astype(jnp.int32), tok_embed, pos_embed)
    return out.reshape(batch, seq_len, d)


# speedup vs baseline: 1.0502x; 1.0123x over previous
import functools

import jax
import jax.numpy as jnp
from jax import lax
from jax.experimental import pallas as pl
from jax.experimental.pallas import tpu as pltpu
from jax.experimental.pallas import tpu_sc as plsc


def _embed_lookup(ids, tok_embed, pos_embed):
    batch, seq_len = ids.shape
    B = batch * seq_len
    _, d = tok_embed.shape
    info = plsc.get_sparse_core_info()
    num_workers = info.num_cores * info.num_subcores
    s_per_w = seq_len // num_workers
    ch = s_per_w // 2
    nch = batch * 2
    depth = 8
    mesh = plsc.VectorSubcoreMesh(core_axis_name="c", subcore_axis_name="s")

    @functools.partial(
        pl.kernel,
        mesh=mesh,
        out_type=jax.ShapeDtypeStruct((B, d), jnp.float32),
        scratch_types=[
            pltpu.VMEM((batch, s_per_w), jnp.int32),
            pltpu.VMEM((nch, ch, d), jnp.float32),
            pltpu.VMEM((s_per_w, d), jnp.float32),
            pltpu.SemaphoreType.DMA((nch,)),
        ],
    )
    def _emb(ids_hbm, tok_hbm, pos_hbm, out_hbm, idx_v, tok_v, pos_v, sem):
        wid = lax.axis_index("s") * info.num_cores + lax.axis_index("c")
        sbase = pl.multiple_of(wid * s_per_w, s_per_w)

        idx_copies = [
            pltpu.async_copy(ids_hbm.at[b, pl.ds(sbase, s_per_w)],
                             idx_v.at[b], sem.at[b])
            for b in range(batch)
        ]
        pos_copy = pltpu.async_copy(pos_hbm.at[pl.ds(sbase, s_per_w)],
                                    pos_v, sem.at[batch])
        for c in idx_copies:
            c.wait()
        pos_copy.wait()

        def prefill(c):
            h = c - 2 * (c // 2)

            @plsc.parallel_loop(0, ch, unroll=2)
            def _row(i):
                for j in range(d // 16):
                    sl = pl.ds(j * 16, 16)
                    tok_v[c, i, sl] = pos_v[h * ch + i, sl]

        def gather(c):
            b = c // 2
            h = c - 2 * b
            return pltpu.make_async_copy(
                tok_hbm.at[idx_v.at[b, pl.ds(h * ch, ch)]],
                tok_v.at[c], sem.at[c])

        def store(c):
            b = c // 2
            h = c - 2 * b
            return pltpu.make_async_copy(
                tok_v.at[c],
                out_hbm.at[pl.ds(b * seq_len + sbase + h * ch, ch)],
                sem.at[c])

        def fire(c, x):
            prefill(c)
            gather(c).start(add=True)
            return x

        lax.fori_loop(0, depth, fire, 0)

        def consume(c, _):
            gather(c).wait()
            store(c).start()

            @pl.when(c + depth < nch)
            def _fire_next():
                fire(c + depth, 0)

            return _

        lax.fori_loop(0, nch, consume, 0)

        lax.fori_loop(0, nch, lambda c, x: (store(c).wait(), x)[1], 0)

    return _emb(ids, tok_embed, pos_embed)


def kernel(ids, tok_embed, pos_embed):
    batch, seq_len = ids.shape
    _, d = tok_embed.shape
    out = _embed_lookup(ids.astype(jnp.int32), tok_embed, pos_embed)
    return out.reshape(batch, seq_len, d)
